# baseline (speedup 1.0000x reference)
import functools

import jax
import jax.numpy as jnp
from jax import lax
from jax.experimental import pallas as pl
from jax.experimental.pallas import tpu as pltpu
from jax.experimental.pallas import tpu_sc as plsc

N = 10000
NP = 10240
E = 160000
B = 2
T = 12
H = 64
PRED = 12
NC = 2
NS = 16
L = 16
CH = 8000
NCH = 20
EP = NCH * CH
EPT = EP // (NC * NS)


def _zero_vmem(buf, n):
    def z(i, _):
        buf[pl.ds(i * L, L)] = jnp.zeros((L,), jnp.float32)
        return 0
    lax.fori_loop(0, n // L, z, 0)


def _deg_body(dst_hbm, ew_hbm, out_hbm, dstb, ewb, zb, idxb, deg_sp):
    c = lax.axis_index("c")
    s = lax.axis_index("s")
    w = c * NS + s
    base = w * EPT
    pltpu.sync_copy(dst_hbm.at[pl.ds(base, EPT)], dstb)
    pltpu.sync_copy(ew_hbm.at[pl.ds(base, EPT)], ewb)
    _zero_vmem(zb, NP)
    pltpu.sync_copy(zb, deg_sp.at[pl.ds(s * NP, NP)])
    soff = s * NP

    def f(i, _):
        dv = dstb[pl.ds(i * L, L)]
        idxb[pl.ds(i * L, L)] = dv + soff
        return 0
    lax.fori_loop(0, EPT // L, f, 0)
    dtail = dstb[pl.ds(EPT - L, L)]
    idxb[pl.ds(EPT - L, L)] = dtail + soff
    pltpu.sync_copy(ewb, deg_sp.at[idxb], add=True)
    pltpu.sync_copy(deg_sp.at[pl.ds(s * NP, NP)], zb)
    pltpu.sync_copy(zb, out_hbm.at[w])


@functools.lru_cache(maxsize=None)
def _deg_call():
    mesh = plsc.VectorSubcoreMesh(core_axis_name="c", subcore_axis_name="s",
                                  num_cores=NC, num_subcores=NS)
    return pl.kernel(
        _deg_body,
        out_type=jax.ShapeDtypeStruct((NC * NS, NP), jnp.float32),
        mesh=mesh,
        compiler_params=pltpu.CompilerParams(needs_layout_passes=False, use_tc_tiling_on_sc=False),
        scratch_types=[
            pltpu.VMEM((EPT,), jnp.int32),
            pltpu.VMEM((EPT,), jnp.float32),
            pltpu.VMEM((NP,), jnp.float32),
            pltpu.VMEM((EPT,), jnp.int32),
            pltpu.VMEM_SHARED((NS * NP,), jnp.float32),
        ],
    )


def _dinv_body(parts_ref, o_ref):
    d = jnp.sum(parts_ref[...], axis=0, keepdims=True) + 1.0
    di = lax.rsqrt(d)
    o_ref[...] = jnp.concatenate([di, di * di], axis=0)


def _dinv_call(parts):
    return pl.pallas_call(
        _dinv_body,
        out_shape=jax.ShapeDtypeStruct((2, NP), jnp.float32),
    )(parts)


def _sconv_body(xt_hbm, dinv_hbm, src_hbm, dst_hbm, ew_hbm, out_hbm,
                xc, dinv, selfb, srcb0, dstb0, ewb0, srcb1, dstb1, ewb1,
                msgb0, idxb0, msgb1, idxb1, sem0, sem1, ssem0, ssem1, s_sp):
    c = lax.axis_index("c")
    s = lax.axis_index("s")

    @pl.when(s < T)
    def _():
        col = c * T + s
        pltpu.sync_copy(xt_hbm.at[col], xc.at[pl.ds(0, N)])

        def zt(i, _):
            xc[pl.ds(N + i * L, L)] = jnp.zeros((L,), jnp.float32)
            return 0
        lax.fori_loop(0, (NP - N) // L, zt, 0)
        pltpu.sync_copy(dinv_hbm.at[0], dinv)

        def yf(i, _):
            dv = dinv[pl.ds(i * L, L)]
            xv = xc[pl.ds(i * L, L)]
            xc[pl.ds(i * L, L)] = dv * xv
            return 0
        lax.fori_loop(0, NP // L, yf, 0)
        soff = s * NP
        pltpu.sync_copy(xc, s_sp.at[pl.ds(soff, NP)])

        srcs = [srcb0, srcb1]
        dsts = [dstb0, dstb1]
        ews = [ewb0, ewb1]
        msgs = [msgb0, msgb1]
        idxs = [idxb0, idxb1]
        sems = [sem0, sem1]
        ssems = [ssem0, ssem1]

        def start_inputs(q, slot):
            base = q * CH
            pltpu.async_copy(src_hbm.at[pl.ds(base, CH)], srcs[slot],
                             sems[slot])
            pltpu.async_copy(dst_hbm.at[pl.ds(base, CH)], dsts[slot],
                             sems[slot])
            pltpu.async_copy(ew_hbm.at[pl.ds(base, CH)], ews[slot],
                             sems[slot])

        def wait_inputs(q, slot):
            base = q * CH
            pltpu.make_async_copy(src_hbm.at[pl.ds(base, CH)], srcs[slot],
                                  sems[slot]).wait()
            pltpu.make_async_copy(dst_hbm.at[pl.ds(base, CH)], dsts[slot],
                                  sems[slot]).wait()
            pltpu.make_async_copy(ew_hbm.at[pl.ds(base, CH)], ews[slot],
                                  sems[slot]).wait()

        start_inputs(0, 0)
        start_inputs(1, 1)

        def chunk2(q2, _):
            for u in range(2):
                q = q2 * 2 + u
                wait_inputs(q, u)

                @pl.when(q >= 2)
                def _():
                    pltpu.make_async_copy(msgs[u], s_sp.at[idxs[u]],
                                          ssems[u]).wait()

                def inner(i4, _):
                    for u4 in range(4):
                        o = (i4 * 4 + u4) * L
                        sv = srcs[u][pl.ds(o, L)]
                        dv = dsts[u][pl.ds(o, L)]
                        ev = ews[u][pl.ds(o, L)]
                        yv = plsc.load_gather(xc, [sv])
                        msgs[u][pl.ds(o, L)] = yv * ev
                        idxs[u][pl.ds(o, L)] = dv + soff
                    return 0
                lax.fori_loop(0, CH // L // 4, inner, 0)

                @pl.when(q + 2 < NCH)
                def _():
                    start_inputs(q + 2, u)
                pltpu.async_copy(msgs[u], s_sp.at[idxs[u]], ssems[u],
                                 add=True)
            return 0
        lax.fori_loop(0, NCH // 2, chunk2, 0)
        pltpu.make_async_copy(msgs[0], s_sp.at[idxs[0]], ssems[0]).wait()
        pltpu.make_async_copy(msgs[1], s_sp.at[idxs[1]], ssems[1]).wait()

        pltpu.sync_copy(s_sp.at[pl.ds(soff, NP)], selfb)

        def ff(i, _):
            av = selfb[pl.ds(i * L, L)]
            dv = dinv[pl.ds(i * L, L)]
            selfb[pl.ds(i * L, L)] = av * dv
            return 0
        lax.fori_loop(0, NP // L, ff, 0)
        pltpu.sync_copy(selfb, out_hbm.at[c * 16 + s])


@functools.lru_cache(maxsize=None)
def _sconv_call():
    mesh = plsc.VectorSubcoreMesh(core_axis_name="c", subcore_axis_name="s",
                                  num_cores=NC, num_subcores=NS)
    return pl.kernel(
        _sconv_body,
        out_type=jax.ShapeDtypeStruct((2 * 16, NP), jnp.float32),
        mesh=mesh,
        compiler_params=pltpu.CompilerParams(needs_layout_passes=False, use_tc_tiling_on_sc=False),
        scratch_types=[
            pltpu.VMEM((NP,), jnp.float32),
            pltpu.VMEM((NP,), jnp.float32),
            pltpu.VMEM((NP,), jnp.float32),
            pltpu.VMEM((CH,), jnp.int32),
            pltpu.VMEM((CH,), jnp.int32),
            pltpu.VMEM((CH,), jnp.float32),
            pltpu.VMEM((CH,), jnp.int32),
            pltpu.VMEM((CH,), jnp.int32),
            pltpu.VMEM((CH,), jnp.float32),
            pltpu.VMEM((CH,), jnp.float32),
            pltpu.VMEM((CH,), jnp.int32),
            pltpu.VMEM((CH,), jnp.float32),
            pltpu.VMEM((CH,), jnp.int32),
            pltpu.SemaphoreType.DMA,
            pltpu.SemaphoreType.DMA,
            pltpu.SemaphoreType.DMA,
            pltpu.SemaphoreType.DMA,
            pltpu.VMEM_SHARED((T * NP,), jnp.float32),
        ],
    )


R = 5120


def _gru_body(s_ref, wg_ref, bg_ref, wih_ref, whh_ref, bih_ref, bhh_ref,
              wo_ref, bo_ref, o_ref):
    s_all = s_ref[...]
    wg = wg_ref[...]
    bg = bg_ref[...]
    bih = bih_ref[...]
    bhh = bhh_ref[...]
    wih = wih_ref[...].astype(jnp.bfloat16)
    whh = whh_ref[...].astype(jnp.bfloat16)
    wo = wo_ref[...].astype(jnp.bfloat16)
    h = jnp.zeros((H, R), jnp.float32)
    for t in range(T):
        st = s_all[t:t + 1, :]
        g = jnp.maximum(wg * st + bg, 0.0)
        gi = jnp.dot(wih, g.astype(jnp.bfloat16),
                     preferred_element_type=jnp.float32) + bih
        gh = jnp.dot(whh, h.astype(jnp.bfloat16),
                     preferred_element_type=jnp.float32) + bhh
        r = jax.nn.sigmoid(gi[0:H] + gh[0:H])
        z = jax.nn.sigmoid(gi[H:2 * H] + gh[H:2 * H])
        n = jnp.tanh(gi[2 * H:] + r * gh[2 * H:])
        h = (1.0 - z) * n + z * h
    o_ref[...] = (jnp.dot(wo, h.astype(jnp.bfloat16),
                          preferred_element_type=jnp.float32) + bo_ref[...])


def _gru_call(s_arr, wg, bg, wih, whh, bih, bhh, wo, bo):
    full = lambda shape: pl.BlockSpec(shape, lambda b, j: (0, 0))
    return pl.pallas_call(
        _gru_body,
        grid=(B, NP // R),
        in_specs=[
            pl.BlockSpec((16, R), lambda b, j: (b, j)),
            full((H, 1)), full((H, 1)),
            full((3 * H, H)), full((3 * H, H)),
            full((3 * H, 1)), full((3 * H, 1)),
            full((16, H)), full((16, 1)),
        ],
        out_specs=pl.BlockSpec((16, R), lambda b, j: (b, j)),
        out_shape=jax.ShapeDtypeStruct((2 * 16, NP), jnp.float32),
    )(s_arr, wg, bg, wih, whh, bih, bhh, wo, bo)


def kernel(x, edge_index, edge_weight, W_gcn, b_gcn, W_ih, W_hh, b_ih, b_hh,
           W_out, b_out):
    src_p = edge_index[0]
    dst_p = edge_index[1]
    ew_p = edge_weight
    xt = jnp.transpose(x[..., 0], (0, 2, 1)).reshape(B * T, N)

    deg_parts = _deg_call()(dst_p, ew_p)
    dinv2 = _dinv_call(deg_parts)
    s_arr = _sconv_call()(xt, dinv2, src_p, dst_p, ew_p)

    wg = W_gcn.reshape(1, H).T
    bg = b_gcn.reshape(H, 1)
    bih = b_ih.reshape(3 * H, 1)
    bhh = b_hh.reshape(3 * H, 1)
    wo = jnp.concatenate(
        [W_out, jnp.zeros((16 - PRED, H), jnp.float32)], axis=0)
    bo = jnp.concatenate(
        [b_out, jnp.zeros((16 - PRED,), jnp.float32)]).reshape(16, 1)

    o = _gru_call(s_arr, wg, bg, W_ih, W_hh, bih, bhh, wo, bo)
    out = o.reshape(B, 16, NP)[:, :PRED, :N].transpose(0, 2, 1)
    return out


# speedup vs baseline: 1.0238x; 1.0238x over previous
import functools

import jax
import jax.numpy as jnp
from jax import lax
from jax.experimental import pallas as pl
from jax.experimental.pallas import tpu as pltpu
from jax.experimental.pallas import tpu_sc as plsc

N = 10000
NP = 10240
E = 160000
B = 2
T = 12
H = 64
PRED = 12
NC = 2
NS = 16
L = 16
CH = 8000
NCH = 20
EP = NCH * CH
EPT = EP // (NC * NS)


def _zero_vmem(buf, n):
    def z(i, _):
        buf[pl.ds(i * L, L)] = jnp.zeros((L,), jnp.float32)
        return 0
    lax.fori_loop(0, n // L, z, 0)


def _deg_body(dst_hbm, ew_hbm, out_hbm, dstb, ewb, zb, idxb, deg_sp):
    c = lax.axis_index("c")
    s = lax.axis_index("s")
    w = c * NS + s
    base = w * EPT
    pltpu.sync_copy(dst_hbm.at[pl.ds(base, EPT)], dstb)
    pltpu.sync_copy(ew_hbm.at[pl.ds(base, EPT)], ewb)
    _zero_vmem(zb, NP)
    pltpu.sync_copy(zb, deg_sp.at[pl.ds(s * NP, NP)])
    soff = s * NP

    def f(i, _):
        dv = dstb[pl.ds(i * L, L)]
        idxb[pl.ds(i * L, L)] = dv + soff
        return 0
    lax.fori_loop(0, EPT // L, f, 0)
    dtail = dstb[pl.ds(EPT - L, L)]
    idxb[pl.ds(EPT - L, L)] = dtail + soff
    pltpu.sync_copy(ewb, deg_sp.at[idxb], add=True)
    pltpu.sync_copy(deg_sp.at[pl.ds(s * NP, NP)], zb)
    pltpu.sync_copy(zb, out_hbm.at[w])


@functools.lru_cache(maxsize=None)
def _deg_call():
    mesh = plsc.VectorSubcoreMesh(core_axis_name="c", subcore_axis_name="s",
                                  num_cores=NC, num_subcores=NS)
    return pl.kernel(
        _deg_body,
        out_type=jax.ShapeDtypeStruct((NC * NS, NP), jnp.float32),
        mesh=mesh,
        compiler_params=pltpu.CompilerParams(needs_layout_passes=False, use_tc_tiling_on_sc=False),
        scratch_types=[
            pltpu.VMEM((EPT,), jnp.int32),
            pltpu.VMEM((EPT,), jnp.float32),
            pltpu.VMEM((NP,), jnp.float32),
            pltpu.VMEM((EPT,), jnp.int32),
            pltpu.VMEM_SHARED((NS * NP,), jnp.float32),
        ],
    )


def _dinv_body(parts_ref, o_ref):
    d = jnp.sum(parts_ref[...], axis=0, keepdims=True) + 1.0
    di = lax.rsqrt(d)
    o_ref[...] = jnp.concatenate([di, di * di], axis=0)


def _dinv_call(parts):
    return pl.pallas_call(
        _dinv_body,
        out_shape=jax.ShapeDtypeStruct((2, NP), jnp.float32),
    )(parts)


def _sconv_body(xt_hbm, dinv_hbm, src_hbm, dst_hbm, ew_hbm, out_hbm,
                xc, dinv, selfb, srcb0, dstb0, ewb0, srcb1, dstb1, ewb1,
                msgb0, idxb0, msgb1, idxb1, sem0, sem1, ssem0, ssem1, s_sp):
    c = lax.axis_index("c")
    s = lax.axis_index("s")

    @pl.when(s < T)
    def _():
        col = c * T + s
        pltpu.sync_copy(xt_hbm.at[col], xc.at[pl.ds(0, N)])

        def zt(i, _):
            xc[pl.ds(N + i * L, L)] = jnp.zeros((L,), jnp.float32)
            return 0
        lax.fori_loop(0, (NP - N) // L, zt, 0)
        pltpu.sync_copy(dinv_hbm.at[0], dinv)

        def yf(i, _):
            dv = dinv[pl.ds(i * L, L)]
            xv = xc[pl.ds(i * L, L)]
            xc[pl.ds(i * L, L)] = dv * xv
            return 0
        lax.fori_loop(0, NP // L, yf, 0)
        soff = s * NP
        pltpu.sync_copy(xc, s_sp.at[pl.ds(soff, NP)])

        srcs = [srcb0, srcb1]
        dsts = [dstb0, dstb1]
        ews = [ewb0, ewb1]
        msgs = [msgb0, msgb1]
        idxs = [idxb0, idxb1]
        sems = [sem0, sem1]
        ssems = [ssem0, ssem1]

        def start_inputs(q, slot):
            base = q * CH
            pltpu.async_copy(src_hbm.at[pl.ds(base, CH)], srcs[slot],
                             sems[slot])
            pltpu.async_copy(dst_hbm.at[pl.ds(base, CH)], dsts[slot],
                             sems[slot])
            pltpu.async_copy(ew_hbm.at[pl.ds(base, CH)], ews[slot],
                             sems[slot])

        def wait_inputs(q, slot):
            base = q * CH
            pltpu.make_async_copy(src_hbm.at[pl.ds(base, CH)], srcs[slot],
                                  sems[slot]).wait()
            pltpu.make_async_copy(dst_hbm.at[pl.ds(base, CH)], dsts[slot],
                                  sems[slot]).wait()
            pltpu.make_async_copy(ew_hbm.at[pl.ds(base, CH)], ews[slot],
                                  sems[slot]).wait()

        start_inputs(0, 0)
        start_inputs(1, 1)

        def chunk2(q2, _):
            for u in range(2):
                q = q2 * 2 + u
                wait_inputs(q, u)

                @pl.when(q >= 2)
                def _():
                    pltpu.make_async_copy(msgs[u], s_sp.at[idxs[u]],
                                          ssems[u]).wait()

                def inner(i4, _):
                    for u4 in range(4):
                        o = (i4 * 4 + u4) * L
                        sv = srcs[u][pl.ds(o, L)]
                        dv = dsts[u][pl.ds(o, L)]
                        ev = ews[u][pl.ds(o, L)]
                        yv = plsc.load_gather(xc, [sv])
                        msgs[u][pl.ds(o, L)] = yv * ev
                        idxs[u][pl.ds(o, L)] = dv + soff
                    return 0
                lax.fori_loop(0, CH // L // 4, inner, 0)

                @pl.when(q + 2 < NCH)
                def _():
                    start_inputs(q + 2, u)
                pltpu.async_copy(msgs[u], s_sp.at[idxs[u]], ssems[u],
                                 add=True)
            return 0
        lax.fori_loop(0, NCH // 2, chunk2, 0)
        pltpu.make_async_copy(msgs[0], s_sp.at[idxs[0]], ssems[0]).wait()
        pltpu.make_async_copy(msgs[1], s_sp.at[idxs[1]], ssems[1]).wait()

        pltpu.sync_copy(s_sp.at[pl.ds(soff, NP)], selfb)

        def ff(i, _):
            av = selfb[pl.ds(i * L, L)]
            dv = dinv[pl.ds(i * L, L)]
            selfb[pl.ds(i * L, L)] = av * dv
            return 0
        lax.fori_loop(0, NP // L, ff, 0)
        pltpu.sync_copy(selfb, out_hbm.at[c * 16 + s])


@functools.lru_cache(maxsize=None)
def _sconv_call():
    mesh = plsc.VectorSubcoreMesh(core_axis_name="c", subcore_axis_name="s",
                                  num_cores=NC, num_subcores=NS)
    return pl.kernel(
        _sconv_body,
        out_type=jax.ShapeDtypeStruct((2 * 16, NP), jnp.float32),
        mesh=mesh,
        compiler_params=pltpu.CompilerParams(needs_layout_passes=False, use_tc_tiling_on_sc=False),
        scratch_types=[
            pltpu.VMEM((NP,), jnp.float32),
            pltpu.VMEM((NP,), jnp.float32),
            pltpu.VMEM((NP,), jnp.float32),
            pltpu.VMEM((CH,), jnp.int32),
            pltpu.VMEM((CH,), jnp.int32),
            pltpu.VMEM((CH,), jnp.float32),
            pltpu.VMEM((CH,), jnp.int32),
            pltpu.VMEM((CH,), jnp.int32),
            pltpu.VMEM((CH,), jnp.float32),
            pltpu.VMEM((CH,), jnp.float32),
            pltpu.VMEM((CH,), jnp.int32),
            pltpu.VMEM((CH,), jnp.float32),
            pltpu.VMEM((CH,), jnp.int32),
            pltpu.SemaphoreType.DMA,
            pltpu.SemaphoreType.DMA,
            pltpu.SemaphoreType.DMA,
            pltpu.SemaphoreType.DMA,
            pltpu.VMEM_SHARED((T * NP,), jnp.float32),
        ],
    )


R = 2048


def _gru_body(s_ref, wg_ref, bg_ref, wih_ref, whh_ref, bih_ref, bhh_ref,
              wo_ref, bo_ref, o_ref):
    s_all = s_ref[...]
    wg = wg_ref[...]
    bg = bg_ref[...]
    bih = bih_ref[...]
    bhh = bhh_ref[...]
    wih = wih_ref[...].astype(jnp.bfloat16)
    whh = whh_ref[...].astype(jnp.bfloat16)
    wo = wo_ref[...].astype(jnp.bfloat16)
    h = jnp.zeros((H, R), jnp.float32)
    for t in range(T):
        st = s_all[t:t + 1, :]
        g = jnp.maximum(wg * st + bg, 0.0)
        gi = jnp.dot(wih, g.astype(jnp.bfloat16),
                     preferred_element_type=jnp.float32) + bih
        gh = jnp.dot(whh, h.astype(jnp.bfloat16),
                     preferred_element_type=jnp.float32) + bhh
        r = jax.nn.sigmoid(gi[0:H] + gh[0:H])
        z = jax.nn.sigmoid(gi[H:2 * H] + gh[H:2 * H])
        n = jnp.tanh(gi[2 * H:] + r * gh[2 * H:])
        h = (1.0 - z) * n + z * h
    o_ref[...] = (jnp.dot(wo, h.astype(jnp.bfloat16),
                          preferred_element_type=jnp.float32) + bo_ref[...])


def _gru_call(s_arr, wg, bg, wih, whh, bih, bhh, wo, bo):
    full = lambda shape: pl.BlockSpec(shape, lambda b, j: (0, 0))
    return pl.pallas_call(
        _gru_body,
        grid=(B, NP // R),
        in_specs=[
            pl.BlockSpec((16, R), lambda b, j: (b, j)),
            full((H, 1)), full((H, 1)),
            full((3 * H, H)), full((3 * H, H)),
            full((3 * H, 1)), full((3 * H, 1)),
            full((16, H)), full((16, 1)),
        ],
        out_specs=pl.BlockSpec((16, R), lambda b, j: (b, j)),
        out_shape=jax.ShapeDtypeStruct((2 * 16, NP), jnp.float32),
    )(s_arr, wg, bg, wih, whh, bih, bhh, wo, bo)


def kernel(x, edge_index, edge_weight, W_gcn, b_gcn, W_ih, W_hh, b_ih, b_hh,
           W_out, b_out):
    src_p = edge_index[0]
    dst_p = edge_index[1]
    ew_p = edge_weight
    xt = jnp.transpose(x[..., 0], (0, 2, 1)).reshape(B * T, N)

    deg_parts = _deg_call()(dst_p, ew_p)
    dinv2 = _dinv_call(deg_parts)
    s_arr = _sconv_call()(xt, dinv2, src_p, dst_p, ew_p)

    wg = W_gcn.reshape(1, H).T
    bg = b_gcn.reshape(H, 1)
    bih = b_ih.reshape(3 * H, 1)
    bhh = b_hh.reshape(3 * H, 1)
    wo = jnp.concatenate(
        [W_out, jnp.zeros((16 - PRED, H), jnp.float32)], axis=0)
    bo = jnp.concatenate(
        [b_out, jnp.zeros((16 - PRED,), jnp.float32)]).reshape(16, 1)

    o = _gru_call(s_arr, wg, bg, W_ih, W_hh, bih, bhh, wo, bo)
    out = o.reshape(B, 16, NP)[:, :PRED, :N].transpose(0, 2, 1)
    return out
